# Initial kernel scaffold; baseline (speedup 1.0000x reference)
import jax
import jax.numpy as jnp
from jax.experimental import pallas as pl


def kernel(pred_vertices, has_smpl, edge):
    raise NotImplementedError("write your pallas kernel here")


# speedup vs baseline: 2.5534x; 2.5534x over previous
import functools

import jax
import jax.numpy as jnp
import numpy as np
from jax import lax
from jax.experimental import pallas as pl
from jax.experimental.pallas import tpu as pltpu
from jax.experimental.pallas import tpu_sc as plsc

_B = 128
_V = 6890
_E = 20664
_NC = 2
_NS = 16
_NW = _NC * _NS
_C = 72
_K = 9
_WPW = _C * _K
_EPAD = _NW * _WPW
_D = 3 * _B
_NG = _B // 16
_EPS = np.float32(1e-8)
_MAGIC = 0x5F3759DF


def _pad_edge_dist() -> np.float32:
    x = np.float32(_EPS)
    bits = x.view(np.int32)
    yi = np.int32(_MAGIC - (int(bits) >> 1))
    y = yi.view(np.float32)
    h = np.float32(0.5) * x
    for _ in range(2):
        y = y * (np.float32(1.5) - h * y * y)
    return np.float32(x * y)


_PAD_DIST = _pad_edge_dist()

_mesh = plsc.VectorSubcoreMesh(core_axis_name="c", subcore_axis_name="s")


@functools.partial(
    pl.kernel,
    mesh=_mesh,
    out_type=jax.ShapeDtypeStruct((_NW, _B), jnp.float32),
    scratch_types=[
        pltpu.VMEM((_WPW,), jnp.int32),
        pltpu.VMEM((_WPW,), jnp.int32),
        pltpu.VMEM((2, _C, _D), jnp.float32),
        pltpu.VMEM((2, _C, _D), jnp.float32),
        pltpu.VMEM((_B,), jnp.float32),
        pltpu.SemaphoreType.DMA,
        pltpu.SemaphoreType.DMA,
        pltpu.SemaphoreType.DMA,
        pltpu.SemaphoreType.DMA,
    ],
)
def _edge_len_partials(e0_hbm, e1_hbm, table_hbm, out_hbm,
                       e0_v, e1_v, rows0, rows1, acc_v,
                       s00, s01, s10, s11):
    wid = lax.axis_index("s") * _NC + lax.axis_index("c")
    base = wid * _WPW
    pltpu.sync_copy(e0_hbm.at[pl.ds(base, _WPW)], e0_v)
    pltpu.sync_copy(e1_hbm.at[pl.ds(base, _WPW)], e1_v)

    sems0 = (s00, s01)
    sems1 = (s10, s11)

    def start(k):
        b = k % 2
        c0 = pltpu.async_copy(
            table_hbm.at[e0_v.at[pl.ds(k * _C, _C)]], rows0.at[b], sems0[b])
        c1 = pltpu.async_copy(
            table_hbm.at[e1_v.at[pl.ds(k * _C, _C)]], rows1.at[b], sems1[b])
        return (c0, c1)

    pending = {0: start(0)}
    acc = tuple(jnp.zeros((16,), jnp.float32) for _ in range(_NG))

    half = jnp.full((16,), 0.5, jnp.float32)
    three_half = jnp.full((16,), 1.5, jnp.float32)
    eps = jnp.full((16,), _EPS, jnp.float32)
    magic = jnp.full((16,), _MAGIC, jnp.int32)

    for k in range(_K):
        if k + 1 < _K:
            pending[k + 1] = start(k + 1)
        for c in pending.pop(k):
            c.wait()
        b = k % 2
        r0 = rows0.at[b]
        r1 = rows1.at[b]

        def body(i, acc):
            out = []
            for g in range(_NG):
                ox, oy, oz = g * 16, _B + g * 16, 2 * _B + g * 16
                dx = r0[i, pl.ds(ox, 16)] - r1[i, pl.ds(ox, 16)]
                dy = r0[i, pl.ds(oy, 16)] - r1[i, pl.ds(oy, 16)]
                dz = r0[i, pl.ds(oz, 16)] - r1[i, pl.ds(oz, 16)]
                ssq = dx * dx + dy * dy + dz * dz + eps
                yi = magic - lax.shift_right_logical(
                    lax.bitcast_convert_type(ssq, jnp.int32), 1)
                y = lax.bitcast_convert_type(yi, jnp.float32)
                h = half * ssq
                y = y * (three_half - h * y * y)
                y = y * (three_half - h * y * y)
                out.append(acc[g] + ssq * y)
            return tuple(out)

        acc = lax.fori_loop(0, _C, body, acc)

    for g in range(_NG):
        acc_v[pl.ds(g * 16, 16)] = acc[g]
    pltpu.sync_copy(acc_v, out_hbm.at[wid])


def kernel(pred_vertices, has_smpl, edge):
    table = jnp.transpose(pred_vertices, (1, 2, 0)).reshape(_V, _D)
    pad = jnp.zeros((_EPAD - _E, 2), jnp.int32)
    ep = jnp.concatenate([edge, pad], axis=0)
    partials = _edge_len_partials(ep[:, 0], ep[:, 1], table)

    mask = (has_smpl == 1).astype(jnp.float32)
    n_sel = jnp.sum(mask)
    per_b = jnp.sum(partials, axis=0)
    total = jnp.sum(per_b * mask)
    total = total - n_sel * np.float32((_EPAD - _E)) * _PAD_DIST
    return total / (n_sel * _E)
